# baseline (speedup 1.0000x reference)
import functools

import jax
import jax.numpy as jnp
from jax.experimental import pallas as pl

_H, _HD = 16, 64


def _kproj_body(wk_ref, xctx_ref, kt_ref):
    kt_ref[...] = jax.lax.dot_general(
        wk_ref[...].astype(jnp.bfloat16), xctx_ref[...].astype(jnp.bfloat16),
        dimension_numbers=(((1,), (1,)), ((), ())),
        preferred_element_type=jnp.float32)


def _score_topk_body(x_ref, k_ref, wq_ref, ww_ref, out_ref, *, heads, hdim, kv):
    xt = x_ref[...]
    tt = xt.shape[0]
    kmat = k_ref[...]
    s = kmat.shape[0]

    q = jax.lax.dot_general(
        xt.astype(jnp.bfloat16), wq_ref[...].astype(jnp.bfloat16),
        dimension_numbers=(((1,), (1,)), ((), ())),
        preferred_element_type=jnp.float32)
    lg = jax.lax.dot_general(
        ww_ref[...].astype(jnp.bfloat16), xt.astype(jnp.bfloat16),
        dimension_numbers=(((1,), (1,)), ((), ())),
        preferred_element_type=jnp.float32)
    lg = lg - jnp.max(lg, axis=0, keepdims=True)
    ex = jnp.exp(lg)
    w = ex / jnp.sum(ex, axis=0, keepdims=True)

    scale = hdim ** -0.5
    acc = jnp.zeros((s, tt), jnp.float32)
    for h in range(heads):
        qh = q[:, h * hdim:(h + 1) * hdim]
        rawt = jax.lax.dot_general(
            kmat.astype(jnp.bfloat16), qh.astype(jnp.bfloat16),
            dimension_numbers=(((1,), (1,)), ((), ())),
            preferred_element_type=jnp.float32)
        rawt = jnp.maximum(rawt * scale, 0.0)
        rawt = rawt.astype(jnp.bfloat16).astype(jnp.float32)
        wh = w[h:h + 1, :].astype(jnp.bfloat16).astype(jnp.float32)
        acc = acc + wh * rawt

    key = acc
    idx = jax.lax.broadcasted_iota(jnp.int32, (s, tt), 0)
    ksz = 2
    while ksz <= s:
        j = ksz // 2
        while j >= 1:
            g = s // (2 * j)
            kr = key.reshape(g, 2 * j, tt)
            ir = idx.reshape(g, 2 * j, tt)
            ka, kb = kr[:, :j, :], kr[:, j:, :]
            ia, ib = ir[:, :j, :], ir[:, j:, :]
            a_gt = (ka > kb) | ((ka == kb) & (ia < ib))
            gi = jax.lax.broadcasted_iota(jnp.int32, (g, 1, 1), 0)
            desc = ((gi * (2 * j)) & ksz) == 0
            swap = jnp.logical_xor(a_gt, desc)
            nka = jnp.where(swap, kb, ka)
            nkb = jnp.where(swap, ka, kb)
            nia = jnp.where(swap, ib, ia)
            nib = jnp.where(swap, ia, ib)
            key = jnp.concatenate([nka, nkb], axis=1).reshape(s, tt)
            idx = jnp.concatenate([nia, nib], axis=1).reshape(s, tt)
            j //= 2
        ksz *= 2

    out_ref[...] = idx[:kv, :]


def _build_calls(t, s, d, heads, hdim, kv, tt, interpret=False):
    kproj = pl.pallas_call(
        _kproj_body,
        out_shape=jax.ShapeDtypeStruct((hdim, s), jnp.float32),
        interpret=interpret,
    )
    body = functools.partial(_score_topk_body, heads=heads, hdim=hdim, kv=kv)
    main = pl.pallas_call(
        body,
        grid=(t // tt,),
        in_specs=[
            pl.BlockSpec((tt, d), lambda i: (i, 0)),
            pl.BlockSpec((s, hdim), lambda i: (0, 0)),
            pl.BlockSpec((heads * hdim, d), lambda i: (0, 0)),
            pl.BlockSpec((heads, d), lambda i: (0, 0)),
        ],
        out_specs=pl.BlockSpec((kv, tt), lambda i: (0, i)),
        out_shape=jax.ShapeDtypeStruct((kv, t), jnp.int32),
        interpret=interpret,
    )
    return kproj, main


def kernel(x, x_ctx, topk, Wq, Wk, Ww):
    b, t, d = x.shape
    s = x_ctx.shape[1]
    kv = min(256, s)
    kproj, main = _build_calls(t, s, d, _H, _HD, kv, tt=256)
    kt = kproj(Wk, x_ctx[0])
    kmat = jnp.swapaxes(kt, 0, 1)
    out = main(x[0], kmat, Wq, Ww)
    return jnp.swapaxes(out, 0, 1)[None]


# speedup vs baseline: 1.2344x; 1.2344x over previous
import functools

import jax
import jax.numpy as jnp
from jax.experimental import pallas as pl

_H, _HD = 16, 64


def _kproj_body(wk_ref, xctx_ref, kt_ref):
    kt_ref[...] = jax.lax.dot_general(
        wk_ref[...].astype(jnp.bfloat16), xctx_ref[...].astype(jnp.bfloat16),
        dimension_numbers=(((1,), (1,)), ((), ())),
        preferred_element_type=jnp.float32)


def _score_topk_body(x_ref, k_ref, wq_ref, ww_ref, out_ref, *, heads, hdim, kv):
    xt = x_ref[...]
    tt = xt.shape[0]
    kmat = k_ref[...]
    s = kmat.shape[0]

    q = jax.lax.dot_general(
        xt.astype(jnp.bfloat16), wq_ref[...].astype(jnp.bfloat16),
        dimension_numbers=(((1,), (1,)), ((), ())),
        preferred_element_type=jnp.float32)
    lg = jax.lax.dot_general(
        ww_ref[...].astype(jnp.bfloat16), xt.astype(jnp.bfloat16),
        dimension_numbers=(((1,), (1,)), ((), ())),
        preferred_element_type=jnp.float32)
    lg = lg - jnp.max(lg, axis=0, keepdims=True)
    ex = jnp.exp(lg)
    w = ex / jnp.sum(ex, axis=0, keepdims=True)

    scale = hdim ** -0.5
    acc = jnp.zeros((s, tt), jnp.float32)
    for h in range(heads):
        qh = q[:, h * hdim:(h + 1) * hdim]
        rawt = jax.lax.dot_general(
            kmat.astype(jnp.bfloat16), qh.astype(jnp.bfloat16),
            dimension_numbers=(((1,), (1,)), ((), ())),
            preferred_element_type=jnp.float32)
        rawt = jnp.maximum(rawt * scale, 0.0)
        rawt = rawt.astype(jnp.bfloat16).astype(jnp.float32)
        wh = w[h:h + 1, :].astype(jnp.bfloat16).astype(jnp.float32)
        acc = acc + wh * rawt

    key = acc
    idx = jax.lax.broadcasted_iota(jnp.int32, (s, tt), 0)
    seg = min(256, s)

    def cmpex(key, idx, length, j, dirbit):
        g = length // (2 * j)
        kr = key.reshape(g, 2 * j, tt)
        ir = idx.reshape(g, 2 * j, tt)
        ka, kb = kr[:, :j, :], kr[:, j:, :]
        ia, ib = ir[:, :j, :], ir[:, j:, :]
        a_gt = (ka > kb) | ((ka == kb) & (ia < ib))
        gi = jax.lax.broadcasted_iota(jnp.int32, (g, 1, 1), 0)
        desc = ((gi * (2 * j)) & dirbit) == 0
        swap = jnp.logical_xor(a_gt, desc)
        nka = jnp.where(swap, kb, ka)
        nkb = jnp.where(swap, ka, kb)
        nia = jnp.where(swap, ib, ia)
        nib = jnp.where(swap, ia, ib)
        key = jnp.concatenate([nka, nkb], axis=1).reshape(length, tt)
        idx = jnp.concatenate([nia, nib], axis=1).reshape(length, tt)
        return key, idx

    ksz = 2
    while ksz <= seg:
        j = ksz // 2
        while j >= 1:
            key, idx = cmpex(key, idx, s, j, ksz)
            j //= 2
        ksz *= 2

    length = s
    while length > seg:
        kr = key.reshape(length // (2 * seg), 2, seg, tt)
        ir = idx.reshape(length // (2 * seg), 2, seg, tt)
        ka, kb = kr[:, 0], kr[:, 1]
        ia, ib = ir[:, 0], ir[:, 1]
        a_gt = (ka > kb) | ((ka == kb) & (ia < ib))
        length //= 2
        key = jnp.where(a_gt, ka, kb).reshape(length, tt)
        idx = jnp.where(a_gt, ia, ib).reshape(length, tt)
        j = seg // 2
        while j >= 1:
            key, idx = cmpex(key, idx, length, j, seg)
            j //= 2

    out_ref[...] = idx[:kv, :]


def _build_calls(t, s, d, heads, hdim, kv, tt, interpret=False):
    kproj = pl.pallas_call(
        _kproj_body,
        out_shape=jax.ShapeDtypeStruct((hdim, s), jnp.float32),
        interpret=interpret,
    )
    body = functools.partial(_score_topk_body, heads=heads, hdim=hdim, kv=kv)
    main = pl.pallas_call(
        body,
        grid=(t // tt,),
        in_specs=[
            pl.BlockSpec((tt, d), lambda i: (i, 0)),
            pl.BlockSpec((s, hdim), lambda i: (0, 0)),
            pl.BlockSpec((heads * hdim, d), lambda i: (0, 0)),
            pl.BlockSpec((heads, d), lambda i: (0, 0)),
        ],
        out_specs=pl.BlockSpec((kv, tt), lambda i: (0, i)),
        out_shape=jax.ShapeDtypeStruct((kv, t), jnp.int32),
        interpret=interpret,
    )
    return kproj, main


def kernel(x, x_ctx, topk, Wq, Wk, Ww):
    b, t, d = x.shape
    s = x_ctx.shape[1]
    kv = min(256, s)
    kproj, main = _build_calls(t, s, d, _H, _HD, kv, tt=256)
    kt = kproj(Wk, x_ctx[0])
    kmat = jnp.swapaxes(kt, 0, 1)
    out = main(x[0], kmat, Wq, Ww)
    return jnp.swapaxes(out, 0, 1)[None]


# speedup vs baseline: 1.2352x; 1.0006x over previous
import functools

import jax
import jax.numpy as jnp
from jax.experimental import pallas as pl

_H, _HD = 16, 64


def _kproj_body(wk_ref, xctx_ref, kt_ref):
    kt_ref[...] = jax.lax.dot_general(
        wk_ref[...].astype(jnp.bfloat16), xctx_ref[...].astype(jnp.bfloat16),
        dimension_numbers=(((1,), (1,)), ((), ())),
        preferred_element_type=jnp.float32)


def _score_topk_body(x_ref, k_ref, wq_ref, ww_ref, out_ref, *, heads, hdim, kv):
    xt = x_ref[...]
    tt = xt.shape[0]
    kmat = k_ref[...]
    s = kmat.shape[0]

    q = jax.lax.dot_general(
        xt.astype(jnp.bfloat16), wq_ref[...].astype(jnp.bfloat16),
        dimension_numbers=(((1,), (1,)), ((), ())),
        preferred_element_type=jnp.float32)
    lg = jax.lax.dot_general(
        ww_ref[...].astype(jnp.bfloat16), xt.astype(jnp.bfloat16),
        dimension_numbers=(((1,), (1,)), ((), ())),
        preferred_element_type=jnp.float32)
    lg = lg - jnp.max(lg, axis=0, keepdims=True)
    ex = jnp.exp(lg)
    w = ex / jnp.sum(ex, axis=0, keepdims=True)

    scale = hdim ** -0.5
    kb = kmat.astype(jnp.bfloat16)
    acc = jnp.zeros((s, tt), jnp.float32)
    for h in range(heads):
        qh = q[:, h * hdim:(h + 1) * hdim]
        rawt = jax.lax.dot_general(
            kb, qh.astype(jnp.bfloat16),
            dimension_numbers=(((1,), (1,)), ((), ())),
            preferred_element_type=jnp.float32)
        rawb = jnp.maximum(rawt.astype(jnp.bfloat16),
                           jnp.bfloat16(0.0)).astype(jnp.float32)
        wh = w[h:h + 1, :].astype(jnp.bfloat16).astype(jnp.float32) * scale
        acc = acc + wh * rawb

    key = acc
    idx = jax.lax.broadcasted_iota(jnp.int32, (s, tt), 0)
    seg = min(256, s)

    def cmpex(key, idx, length, j, dirbit):
        g = length // (2 * j)
        kr = key.reshape(g, 2 * j, tt)
        ir = idx.reshape(g, 2 * j, tt)
        ka, kb = kr[:, :j, :], kr[:, j:, :]
        ia, ib = ir[:, :j, :], ir[:, j:, :]
        a_gt = (ka > kb) | ((ka == kb) & (ia < ib))
        gi = jax.lax.broadcasted_iota(jnp.int32, (g, 1, 1), 0)
        desc = ((gi * (2 * j)) & dirbit) == 0
        swap = jnp.logical_xor(a_gt, desc)
        nka = jnp.where(swap, kb, ka)
        nkb = jnp.where(swap, ka, kb)
        nia = jnp.where(swap, ib, ia)
        nib = jnp.where(swap, ia, ib)
        key = jnp.concatenate([nka, nkb], axis=1).reshape(length, tt)
        idx = jnp.concatenate([nia, nib], axis=1).reshape(length, tt)
        return key, idx

    ksz = 2
    while ksz <= seg:
        j = ksz // 2
        while j >= 1:
            key, idx = cmpex(key, idx, s, j, ksz)
            j //= 2
        ksz *= 2

    length = s
    while length > seg:
        kr = key.reshape(length // (2 * seg), 2, seg, tt)
        ir = idx.reshape(length // (2 * seg), 2, seg, tt)
        ka, kb = kr[:, 0], kr[:, 1]
        ia, ib = ir[:, 0], ir[:, 1]
        a_gt = (ka > kb) | ((ka == kb) & (ia < ib))
        length //= 2
        key = jnp.where(a_gt, ka, kb).reshape(length, tt)
        idx = jnp.where(a_gt, ia, ib).reshape(length, tt)
        j = seg // 2
        while j >= 1:
            key, idx = cmpex(key, idx, length, j, seg)
            j //= 2

    out_ref[...] = idx[:kv, :]


def _build_calls(t, s, d, heads, hdim, kv, tt, interpret=False):
    kproj = pl.pallas_call(
        _kproj_body,
        out_shape=jax.ShapeDtypeStruct((hdim, s), jnp.float32),
        interpret=interpret,
    )
    body = functools.partial(_score_topk_body, heads=heads, hdim=hdim, kv=kv)
    main = pl.pallas_call(
        body,
        grid=(t // tt,),
        in_specs=[
            pl.BlockSpec((tt, d), lambda i: (i, 0)),
            pl.BlockSpec((s, hdim), lambda i: (0, 0)),
            pl.BlockSpec((heads * hdim, d), lambda i: (0, 0)),
            pl.BlockSpec((heads, d), lambda i: (0, 0)),
        ],
        out_specs=pl.BlockSpec((kv, tt), lambda i: (0, i)),
        out_shape=jax.ShapeDtypeStruct((kv, t), jnp.int32),
        interpret=interpret,
    )
    return kproj, main


def kernel(x, x_ctx, topk, Wq, Wk, Ww):
    b, t, d = x.shape
    s = x_ctx.shape[1]
    kv = min(256, s)
    kproj, main = _build_calls(t, s, d, _H, _HD, kv, tt=256)
    kt = kproj(Wk, x_ctx[0])
    kmat = jnp.swapaxes(kt, 0, 1)
    out = main(x[0], kmat, Wq, Ww)
    return jnp.swapaxes(out, 0, 1)[None]


# speedup vs baseline: 1.4565x; 1.1792x over previous
import functools

import jax
import jax.numpy as jnp
from jax.experimental import pallas as pl

_H, _HD = 16, 64


def _kproj_body(wk_ref, xctx_ref, kt_ref):
    kt_ref[...] = jax.lax.dot_general(
        wk_ref[...].astype(jnp.bfloat16), xctx_ref[...].astype(jnp.bfloat16),
        dimension_numbers=(((1,), (1,)), ((), ())),
        preferred_element_type=jnp.float32)


def _score_topk_body(x_ref, k_ref, wq_ref, ww_ref, out_ref, *, heads, hdim, kv):
    xt = x_ref[...]
    tt = xt.shape[0]
    kmat = k_ref[...]
    s = kmat.shape[0]

    q = jax.lax.dot_general(
        xt.astype(jnp.bfloat16), wq_ref[...].astype(jnp.bfloat16),
        dimension_numbers=(((1,), (1,)), ((), ())),
        preferred_element_type=jnp.float32)
    lg = jax.lax.dot_general(
        ww_ref[...].astype(jnp.bfloat16), xt.astype(jnp.bfloat16),
        dimension_numbers=(((1,), (1,)), ((), ())),
        preferred_element_type=jnp.float32)
    lg = lg - jnp.max(lg, axis=0, keepdims=True)
    ex = jnp.exp(lg)
    w = ex / jnp.sum(ex, axis=0, keepdims=True)

    scale = hdim ** -0.5
    kb = kmat.astype(jnp.bfloat16)
    acc = jnp.zeros((s, tt), jnp.float32)
    for h in range(heads):
        qh = q[:, h * hdim:(h + 1) * hdim]
        rawt = jax.lax.dot_general(
            kb, qh.astype(jnp.bfloat16),
            dimension_numbers=(((1,), (1,)), ((), ())),
            preferred_element_type=jnp.float32)
        rawb = jnp.maximum(rawt.astype(jnp.bfloat16),
                           jnp.bfloat16(0.0)).astype(jnp.float32)
        wh = w[h:h + 1, :].astype(jnp.bfloat16).astype(jnp.float32) * scale
        acc = acc + wh * rawb

    key = acc
    idx = jax.lax.broadcasted_iota(jnp.int32, (s, tt), 0)
    seg = min(256, s)

    def cmpex(key, idx, length, j, dirbit):
        g = length // (2 * j)
        kr = key.reshape(g, 2 * j, tt)
        ir = idx.reshape(g, 2 * j, tt)
        ka, kb = kr[:, :j, :], kr[:, j:, :]
        ia, ib = ir[:, :j, :], ir[:, j:, :]
        a_gt = (ka > kb) | ((ka == kb) & (ia < ib))
        gi = jax.lax.broadcasted_iota(jnp.int32, (g, 1, 1), 0)
        desc = ((gi * (2 * j)) & dirbit) == 0
        swap = jnp.logical_xor(a_gt, desc)
        nka = jnp.where(swap, kb, ka)
        nkb = jnp.where(swap, ka, kb)
        nia = jnp.where(swap, ib, ia)
        nib = jnp.where(swap, ia, ib)
        key = jnp.concatenate([nka, nkb], axis=1).reshape(length, tt)
        idx = jnp.concatenate([nia, nib], axis=1).reshape(length, tt)
        return key, idx

    ksz = 2
    while ksz <= seg:
        j = ksz // 2
        while j >= 1:
            key, idx = cmpex(key, idx, s, j, ksz)
            j //= 2
        ksz *= 2

    length = s
    while length > seg:
        kr = key.reshape(length // (2 * seg), 2, seg, tt)
        ir = idx.reshape(length // (2 * seg), 2, seg, tt)
        ka, kb = kr[:, 0], kr[:, 1]
        ia, ib = ir[:, 0], ir[:, 1]
        a_gt = (ka > kb) | ((ka == kb) & (ia < ib))
        length //= 2
        key = jnp.where(a_gt, ka, kb).reshape(length, tt)
        idx = jnp.where(a_gt, ia, ib).reshape(length, tt)
        j = seg // 2
        while j >= 1:
            key, idx = cmpex(key, idx, length, j, seg)
            j //= 2

    out_ref[...] = idx[:kv, :]


def _build_calls(t, s, d, heads, hdim, kv, tt, interpret=False):
    kproj = pl.pallas_call(
        _kproj_body,
        out_shape=jax.ShapeDtypeStruct((hdim, s), jnp.float32),
        interpret=interpret,
    )
    body = functools.partial(_score_topk_body, heads=heads, hdim=hdim, kv=kv)
    main = pl.pallas_call(
        body,
        grid=(t // tt,),
        in_specs=[
            pl.BlockSpec((tt, d), lambda i: (i, 0)),
            pl.BlockSpec((s, hdim), lambda i: (0, 0)),
            pl.BlockSpec((heads * hdim, d), lambda i: (0, 0)),
            pl.BlockSpec((heads, d), lambda i: (0, 0)),
        ],
        out_specs=pl.BlockSpec((kv, tt), lambda i: (0, i)),
        out_shape=jax.ShapeDtypeStruct((kv, t), jnp.int32),
        interpret=interpret,
    )
    return kproj, main


def kernel(x, x_ctx, topk, Wq, Wk, Ww):
    b, t, d = x.shape
    s = x_ctx.shape[1]
    kv = min(256, s)
    kproj, main = _build_calls(t, s, d, _H, _HD, kv, tt=512)
    kt = kproj(Wk, x_ctx[0])
    kmat = jnp.swapaxes(kt, 0, 1)
    out = main(x[0], kmat, Wq, Ww)
    return jnp.swapaxes(out, 0, 1)[None]
